# Initial kernel scaffold; baseline (speedup 1.0000x reference)
import jax
import jax.numpy as jnp
from jax.experimental import pallas as pl


def kernel(x, adj, params):
    raise NotImplementedError("write your pallas kernel here")


# speedup vs baseline: 26.7845x; 26.7845x over previous
import jax
import jax.numpy as jnp
from jax import lax
from jax.experimental import pallas as pl

N = 4096
K = 10
MU = 0.5
EPS_NORM = 1e-12
EPS_DEG = 1e-12
NEG = -3.0e38

BJ = 256
BAE = 512

AE_ORDER = ["enc1", "enc2", "enc3", "zl", "dec1", "dec2", "dec3", "xbar"]
AE_RELU = [True, True, True, False, True, True, True, False]


def _dotT(a, b):
    return lax.dot_general(a, b, (((1,), (1,)), ((), ())),
                           preferred_element_type=jnp.float32)


def _dotTL(a, b):
    return lax.dot_general(a, b, (((0,), (0,)), ((), ())),
                           preferred_element_type=jnp.float32)


def _topk_thr(s):
    cur = s
    for _ in range(K - 1):
        m = jnp.max(cur, axis=1, keepdims=True)
        cur = jnp.where(cur >= m, NEG, cur)
    return jnp.max(cur, axis=1, keepdims=True)


def _ae_body(x_ref, *refs):
    w_refs = refs[0:8]
    b_refs = refs[8:16]
    g1_ref = refs[16]
    outs = refs[17:]
    h = x_ref[...]
    hs = []
    for wr, br, rl in zip(w_refs, b_refs, AE_RELU):
        h = jnp.dot(h, wr[...], preferred_element_type=jnp.float32) + br[...]
        if rl:
            h = jax.nn.relu(h)
        hs.append(h)
    outs[0][...] = hs[7]
    outs[1][...] = hs[0]
    outs[2][...] = hs[1]
    outs[3][...] = hs[2]
    outs[4][...] = hs[3]
    outs[5][...] = hs[4]
    outs[6][...] = hs[5]
    outs[7][...] = hs[6]
    outs[8][...] = jnp.dot(x_ref[...], g1_ref[...],
                           preferred_element_type=jnp.float32)


def _run_ae(x, p):
    dims = [128, 64, 32, 16, 32, 64, 128]
    in_specs = [pl.BlockSpec((BAE, 128), lambda i: (i, 0))]
    args = [x]
    for nm in AE_ORDER:
        w = p[nm + "_w"]
        in_specs.append(pl.BlockSpec(w.shape, lambda i: (0, 0)))
        args.append(w)
    for nm in AE_ORDER:
        b = p[nm + "_b"].reshape(1, -1)
        in_specs.append(pl.BlockSpec(b.shape, lambda i: (0, 0)))
        args.append(b)
    in_specs.append(pl.BlockSpec((128, 128), lambda i: (0, 0)))
    args.append(p["g1_w"])
    out_dims = [128] + dims + [128]
    out_shapes = [jax.ShapeDtypeStruct((N, d), jnp.float32) for d in out_dims]
    out_specs = [pl.BlockSpec((BAE, d), lambda i: (i, 0)) for d in out_dims]
    return pl.pallas_call(
        _ae_body,
        grid=(N // BAE,),
        in_specs=in_specs,
        out_specs=out_specs,
        out_shape=out_shapes,
    )(*args)


def _l1_body(adj_ref, sup1_ref, t1_ref, f1_ref, g2_ref,
             h_ref, hn_ref, sup2_ref, arc_ref, arr_ref):
    i = pl.program_id(0)
    acc = jnp.dot(adj_ref[...], sup1_ref[...],
                  preferred_element_type=jnp.float32)
    g = jax.nn.sigmoid(f1_ref[...])
    h = g * jax.nn.relu(acc) + (1.0 - g) * t1_ref[...]
    nrm = jnp.sqrt(jnp.sum(h * h, axis=1, keepdims=True)) + EPS_NORM
    h_ref[...] = h
    hn_ref[...] = h / nrm
    sup2_ref[...] = jnp.dot(h, g2_ref[...], preferred_element_type=jnp.float32)
    rs = jnp.sum(adj_ref[...], axis=1, keepdims=True)
    arc_ref[...] = rs
    arr_ref[0, pl.ds(i * BJ, BJ)] = rs[:, 0]


def _run_l1(adj, sup1, t1, f1w, g2w):
    f2o = g2w.shape[1]
    return pl.pallas_call(
        _l1_body,
        grid=(N // BJ,),
        in_specs=[
            pl.BlockSpec((BJ, N), lambda i: (i, 0)),
            pl.BlockSpec((N, 128), lambda i: (0, 0)),
            pl.BlockSpec((BJ, 128), lambda i: (i, 0)),
            pl.BlockSpec((1, 128), lambda i: (0, 0)),
            pl.BlockSpec((128, f2o), lambda i: (0, 0)),
        ],
        out_specs=[
            pl.BlockSpec((BJ, 128), lambda i: (i, 0)),
            pl.BlockSpec((BJ, 128), lambda i: (i, 0)),
            pl.BlockSpec((BJ, f2o), lambda i: (i, 0)),
            pl.BlockSpec((BJ, 1), lambda i: (i, 0)),
            pl.BlockSpec((1, N), lambda i: (0, 0)),
        ],
        out_shape=[
            jax.ShapeDtypeStruct((N, 128), jnp.float32),
            jax.ShapeDtypeStruct((N, 128), jnp.float32),
            jax.ShapeDtypeStruct((N, f2o), jnp.float32),
            jax.ShapeDtypeStruct((N, 1), jnp.float32),
            jax.ShapeDtypeStruct((1, N), jnp.float32),
        ],
    )(adj, sup1, t1, f1w, g2w)


def _p1_body(hnj_ref, hn_ref, arc_ref, thr_ref, degc_ref, degr_ref):
    i = pl.program_id(0)
    s = _dotT(hnj_ref[...], hn_ref[...])
    thr = _topk_thr(s)
    st = jnp.where(jnp.logical_and(s >= thr, s > 0.0), s, 0.0)
    srow = jnp.sum(st, axis=1, keepdims=True)
    deg = 1.0 / jnp.sqrt((1.0 - MU) * arc_ref[...] + MU * srow + EPS_DEG)
    thr_ref[...] = thr
    degc_ref[...] = deg
    degr_ref[0, pl.ds(i * BJ, BJ)] = deg[:, 0]


def _run_p1(hn, adjrow_c):
    f = hn.shape[1]
    return pl.pallas_call(
        _p1_body,
        grid=(N // BJ,),
        in_specs=[
            pl.BlockSpec((BJ, f), lambda i: (i, 0)),
            pl.BlockSpec((N, f), lambda i: (0, 0)),
            pl.BlockSpec((BJ, 1), lambda i: (i, 0)),
        ],
        out_specs=[
            pl.BlockSpec((BJ, 1), lambda i: (i, 0)),
            pl.BlockSpec((BJ, 1), lambda i: (i, 0)),
            pl.BlockSpec((1, N), lambda i: (0, 0)),
        ],
        out_shape=[
            jax.ShapeDtypeStruct((N, 1), jnp.float32),
            jax.ShapeDtypeStruct((N, 1), jnp.float32),
            jax.ShapeDtypeStruct((1, N), jnp.float32),
        ],
    )(hn, hn, adjrow_c)


def _p2_body(adj_ref, hnj_ref, hn_ref, supj_ref, thr_ref, degc_ref, degr_ref,
             hacc_ref):
    j = pl.program_id(0)
    s = _dotT(hnj_ref[...], hn_ref[...])
    thr = thr_ref[...]
    st = jnp.where(jnp.logical_and(s >= thr, s > 0.0), s, 0.0)
    w = (1.0 - MU) * adj_ref[...] + MU * st
    a = (w * degr_ref[...]) * degc_ref[...]
    contrib = _dotTL(a, supj_ref[...])

    @pl.when(j == 0)
    def _():
        hacc_ref[...] = jnp.zeros_like(hacc_ref)

    hacc_ref[...] += contrib


def _run_p2(adj, hn, sup, thr, degc, degr):
    f = hn.shape[1]
    fo = sup.shape[1]
    return pl.pallas_call(
        _p2_body,
        grid=(N // BJ,),
        in_specs=[
            pl.BlockSpec((BJ, N), lambda j: (j, 0)),
            pl.BlockSpec((BJ, f), lambda j: (j, 0)),
            pl.BlockSpec((N, f), lambda j: (0, 0)),
            pl.BlockSpec((BJ, fo), lambda j: (j, 0)),
            pl.BlockSpec((BJ, 1), lambda j: (j, 0)),
            pl.BlockSpec((BJ, 1), lambda j: (j, 0)),
            pl.BlockSpec((1, N), lambda j: (0, 0)),
        ],
        out_specs=pl.BlockSpec((N, fo), lambda j: (0, 0)),
        out_shape=jax.ShapeDtypeStruct((N, fo), jnp.float32),
    )(adj, hn, hn, sup, thr, degc, degr)


def _epi_body(hacc_ref, t_ref, fw_ref, wn_ref, h_ref, hn_ref, sup_ref):
    g = jax.nn.sigmoid(fw_ref[...])
    h = g * jax.nn.relu(hacc_ref[...]) + (1.0 - g) * t_ref[...]
    nrm = jnp.sqrt(jnp.sum(h * h, axis=1, keepdims=True)) + EPS_NORM
    h_ref[...] = h
    hn_ref[...] = h / nrm
    if sup_ref is not None:
        sup_ref[...] = jnp.dot(h, wn_ref[...],
                               preferred_element_type=jnp.float32)


def _epi_body_last(hacc_ref, t_ref, fw_ref, h_ref, hn_ref):
    _epi_body(hacc_ref, t_ref, fw_ref, None, h_ref, hn_ref, None)


def _run_epi(hacc, t, fw, wn):
    f = hacc.shape[1]
    in_specs = [
        pl.BlockSpec((BAE, f), lambda i: (i, 0)),
        pl.BlockSpec((BAE, f), lambda i: (i, 0)),
        pl.BlockSpec((1, f), lambda i: (0, 0)),
    ]
    args = [hacc, t, fw]
    out_shapes = [jax.ShapeDtypeStruct((N, f), jnp.float32),
                  jax.ShapeDtypeStruct((N, f), jnp.float32)]
    out_specs = [pl.BlockSpec((BAE, f), lambda i: (i, 0)),
                 pl.BlockSpec((BAE, f), lambda i: (i, 0))]
    if wn is not None:
        fo = wn.shape[1]
        in_specs.append(pl.BlockSpec((f, fo), lambda i: (0, 0)))
        args.append(wn)
        out_shapes.append(jax.ShapeDtypeStruct((N, fo), jnp.float32))
        out_specs.append(pl.BlockSpec((BAE, fo), lambda i: (i, 0)))
        body = _epi_body
    else:
        body = _epi_body_last
    return pl.pallas_call(
        body,
        grid=(N // BAE,),
        in_specs=in_specs,
        out_specs=out_specs,
        out_shape=out_shapes,
    )(*args)


def _apred_body(adj_ref, hnj_ref, hn_ref, a_ref):
    s = _dotT(hnj_ref[...], hn_ref[...])
    thr = _topk_thr(s)
    st = jnp.where(jnp.logical_and(s >= thr, s > 0.0), s, 0.0)
    a_ref[...] = (1.0 - MU) * adj_ref[...] + MU * st


def _run_apred(adj, hn):
    return pl.pallas_call(
        _apred_body,
        grid=(N // BJ,),
        in_specs=[
            pl.BlockSpec((BJ, N), lambda j: (j, 0)),
            pl.BlockSpec((BJ, 128), lambda j: (j, 0)),
            pl.BlockSpec((N, 128), lambda j: (0, 0)),
        ],
        out_specs=pl.BlockSpec((BJ, N), lambda j: (j, 0)),
        out_shape=jax.ShapeDtypeStruct((N, N), jnp.float32),
    )(adj, hn, hn)


def kernel(x, adj, params):
    p = params
    (x_bar, t1, t2, t3, z, d1, d2, d3, sup1) = _run_ae(x, p)

    fws = [p["f%d_w" % i].reshape(1, -1) for i in range(1, 9)]
    gws = [p["g%d_w" % i] for i in range(1, 9)]
    fuse_t = [t1, t2, t3, z, d1, d2, d3, x_bar]

    h, hn, sup, adjrow_c, adjrow_r = _run_l1(adj, sup1, t1, fws[0], gws[1])

    h1_saved = None
    for li in range(2, 9):
        thr, degc, degr = _run_p1(hn, adjrow_c)
        hacc = _run_p2(adj, hn, sup, thr, degc, degr)
        wn = gws[li] if li < 8 else None
        res = _run_epi(hacc, fuse_t[li - 1], fws[li - 1], wn)
        if li < 8:
            h, hn, sup = res
        else:
            h, hn = res
        if li == 4:
            h1_saved = h

    a_pred = _run_apred(adj, hn)
    return (x_bar, z, a_pred, h, h1_saved)


# speedup vs baseline: 27.0839x; 1.0112x over previous
import jax
import jax.numpy as jnp
from jax import lax
from jax.experimental import pallas as pl
from jax.experimental.pallas import tpu as pltpu

N = 4096
K = 10
MU = 0.5
EPS_NORM = 1e-12
EPS_DEG = 1e-12
NEG = -3.0e38

BJ = 256
BAE = 512

AE_ORDER = ["enc1", "enc2", "enc3", "zl", "dec1", "dec2", "dec3", "xbar"]
AE_RELU = [True, True, True, False, True, True, True, False]


def _dotT(a, b):
    return lax.dot_general(a, b, (((1,), (1,)), ((), ())),
                           preferred_element_type=jnp.float32)


def _dotTL(a, b):
    return lax.dot_general(a, b, (((0,), (0,)), ((), ())),
                           preferred_element_type=jnp.float32)


def _topk_thr(s):
    cur = s
    for _ in range(K - 1):
        m = jnp.max(cur, axis=1, keepdims=True)
        cur = jnp.where(cur >= m, NEG, cur)
    return jnp.max(cur, axis=1, keepdims=True)


def _ae_body(x_ref, *refs):
    w_refs = refs[0:8]
    b_refs = refs[8:16]
    g1_ref = refs[16]
    outs = refs[17:]
    h = x_ref[...]
    hs = []
    for wr, br, rl in zip(w_refs, b_refs, AE_RELU):
        h = jnp.dot(h, wr[...], preferred_element_type=jnp.float32) + br[...]
        if rl:
            h = jax.nn.relu(h)
        hs.append(h)
    outs[0][...] = hs[7]
    outs[1][...] = hs[0]
    outs[2][...] = hs[1]
    outs[3][...] = hs[2]
    outs[4][...] = hs[3]
    outs[5][...] = hs[4]
    outs[6][...] = hs[5]
    outs[7][...] = hs[6]
    outs[8][...] = jnp.dot(x_ref[...], g1_ref[...],
                           preferred_element_type=jnp.float32)


def _run_ae(x, p):
    dims = [128, 64, 32, 16, 32, 64, 128]
    in_specs = [pl.BlockSpec((BAE, 128), lambda i: (i, 0))]
    args = [x]
    for nm in AE_ORDER:
        w = p[nm + "_w"]
        in_specs.append(pl.BlockSpec(w.shape, lambda i: (0, 0)))
        args.append(w)
    for nm in AE_ORDER:
        b = p[nm + "_b"].reshape(1, -1)
        in_specs.append(pl.BlockSpec(b.shape, lambda i: (0, 0)))
        args.append(b)
    in_specs.append(pl.BlockSpec((128, 128), lambda i: (0, 0)))
    args.append(p["g1_w"])
    out_dims = [128] + dims + [128]
    out_shapes = [jax.ShapeDtypeStruct((N, d), jnp.float32) for d in out_dims]
    out_specs = [pl.BlockSpec((BAE, d), lambda i: (i, 0)) for d in out_dims]
    return pl.pallas_call(
        _ae_body,
        grid=(N // BAE,),
        in_specs=in_specs,
        out_specs=out_specs,
        out_shape=out_shapes,
    )(*args)


def _l1_body(adj_ref, sup1_ref, t1_ref, f1_ref, g2_ref,
             h_ref, hn_ref, sup2_ref, arc_ref, arr_ref):
    i = pl.program_id(0)
    acc = jnp.dot(adj_ref[...], sup1_ref[...],
                  preferred_element_type=jnp.float32)
    g = jax.nn.sigmoid(f1_ref[...])
    h = g * jax.nn.relu(acc) + (1.0 - g) * t1_ref[...]
    nrm = jnp.sqrt(jnp.sum(h * h, axis=1, keepdims=True)) + EPS_NORM
    h_ref[...] = h
    hn_ref[...] = h / nrm
    sup2_ref[...] = jnp.dot(h, g2_ref[...], preferred_element_type=jnp.float32)
    rs = jnp.sum(adj_ref[...], axis=1, keepdims=True)
    arc_ref[...] = rs
    arr_ref[0, pl.ds(i * BJ, BJ)] = rs[:, 0]


def _run_l1(adj, sup1, t1, f1w, g2w):
    f2o = g2w.shape[1]
    return pl.pallas_call(
        _l1_body,
        grid=(N // BJ,),
        in_specs=[
            pl.BlockSpec((BJ, N), lambda i: (i, 0)),
            pl.BlockSpec((N, 128), lambda i: (0, 0)),
            pl.BlockSpec((BJ, 128), lambda i: (i, 0)),
            pl.BlockSpec((1, 128), lambda i: (0, 0)),
            pl.BlockSpec((128, f2o), lambda i: (0, 0)),
        ],
        out_specs=[
            pl.BlockSpec((BJ, 128), lambda i: (i, 0)),
            pl.BlockSpec((BJ, 128), lambda i: (i, 0)),
            pl.BlockSpec((BJ, f2o), lambda i: (i, 0)),
            pl.BlockSpec((BJ, 1), lambda i: (i, 0)),
            pl.BlockSpec((1, N), lambda i: (0, 0)),
        ],
        out_shape=[
            jax.ShapeDtypeStruct((N, 128), jnp.float32),
            jax.ShapeDtypeStruct((N, 128), jnp.float32),
            jax.ShapeDtypeStruct((N, f2o), jnp.float32),
            jax.ShapeDtypeStruct((N, 1), jnp.float32),
            jax.ShapeDtypeStruct((1, N), jnp.float32),
        ],
    )(adj, sup1, t1, f1w, g2w)


_G = N // BJ


def _layer_body(adj_ref, hnj_ref, hn_ref, supj_ref, arc_ref, t_ref,
                fw_ref, wn_ref, h_ref, hno_ref, supo_ref,
                thr_s, degc_s, degr_s, hacc_s):
    t = pl.program_id(0)
    i = lax.rem(t, _G)
    s = _dotT(hnj_ref[...], hn_ref[...])

    @pl.when(t < _G)
    def _p1():
        thr = _topk_thr(s)
        st = jnp.where(jnp.logical_and(s >= thr, s > 0.0), s, 0.0)
        srow = jnp.sum(st, axis=1, keepdims=True)
        deg = 1.0 / jnp.sqrt((1.0 - MU) * arc_ref[...] + MU * srow + EPS_DEG)
        thr_s[pl.ds(i * BJ, BJ), :] = thr
        degc_s[pl.ds(i * BJ, BJ), :] = deg
        degr_s[0, pl.ds(i * BJ, BJ)] = deg[:, 0]

    @pl.when(t >= _G)
    def _p2():
        thr = thr_s[pl.ds(i * BJ, BJ), :]
        st = jnp.where(jnp.logical_and(s >= thr, s > 0.0), s, 0.0)
        w = (1.0 - MU) * adj_ref[...] + MU * st
        a = (w * degr_s[...]) * degc_s[pl.ds(i * BJ, BJ), :]
        contrib = _dotTL(a, supj_ref[...])

        @pl.when(t == _G)
        def _():
            hacc_s[...] = jnp.zeros_like(hacc_s)

        hacc_s[...] += contrib

    @pl.when(t == 2 * _G - 1)
    def _epi():
        g = jax.nn.sigmoid(fw_ref[...])
        h = g * jax.nn.relu(hacc_s[...]) + (1.0 - g) * t_ref[...]
        nrm = jnp.sqrt(jnp.sum(h * h, axis=1, keepdims=True)) + EPS_NORM
        h_ref[...] = h
        hno_ref[...] = h / nrm
        if supo_ref is not None:
            supo_ref[...] = jnp.dot(h, wn_ref[...],
                                    preferred_element_type=jnp.float32)


def _layer_body_last(adj_ref, hnj_ref, hn_ref, supj_ref, arc_ref, t_ref,
                     fw_ref, h_ref, hno_ref, thr_s, degc_s, degr_s, hacc_s):
    _layer_body(adj_ref, hnj_ref, hn_ref, supj_ref, arc_ref, t_ref,
                fw_ref, None, h_ref, hno_ref, None,
                thr_s, degc_s, degr_s, hacc_s)


def _run_layer(adj, hn, sup, arc, t_fuse, fw, wn):
    f = hn.shape[1]
    fo = sup.shape[1]
    in_specs = [
        pl.BlockSpec((BJ, N), lambda t: (jnp.maximum(t - _G, 0), 0)),
        pl.BlockSpec((BJ, f), lambda t: (lax.rem(t, _G), 0)),
        pl.BlockSpec((N, f), lambda t: (0, 0)),
        pl.BlockSpec((BJ, fo), lambda t: (lax.rem(t, _G), 0)),
        pl.BlockSpec((BJ, 1), lambda t: (lax.rem(t, _G), 0)),
        pl.BlockSpec((N, fo), lambda t: (0, 0)),
        pl.BlockSpec((1, fo), lambda t: (0, 0)),
    ]
    args = [adj, hn, hn, sup, arc, t_fuse, fw]
    out_shapes = [jax.ShapeDtypeStruct((N, fo), jnp.float32),
                  jax.ShapeDtypeStruct((N, fo), jnp.float32)]
    out_specs = [pl.BlockSpec((N, fo), lambda t: (0, 0)),
                 pl.BlockSpec((N, fo), lambda t: (0, 0))]
    if wn is not None:
        fo2 = wn.shape[1]
        in_specs.append(pl.BlockSpec((f if False else wn.shape[0], fo2),
                                     lambda t: (0, 0)))
        args.append(wn)
        out_shapes.append(jax.ShapeDtypeStruct((N, fo2), jnp.float32))
        out_specs.append(pl.BlockSpec((N, fo2), lambda t: (0, 0)))
        body = _layer_body
    else:
        body = _layer_body_last
    return pl.pallas_call(
        body,
        grid=(2 * _G,),
        in_specs=in_specs,
        out_specs=out_specs,
        out_shape=out_shapes,
        scratch_shapes=[
            pltpu.VMEM((N, 1), jnp.float32),
            pltpu.VMEM((N, 1), jnp.float32),
            pltpu.VMEM((1, N), jnp.float32),
            pltpu.VMEM((N, fo), jnp.float32),
        ],
    )(*args)


def _apred_body(adj_ref, hnj_ref, hn_ref, a_ref):
    s = _dotT(hnj_ref[...], hn_ref[...])
    thr = _topk_thr(s)
    st = jnp.where(jnp.logical_and(s >= thr, s > 0.0), s, 0.0)
    a_ref[...] = (1.0 - MU) * adj_ref[...] + MU * st


def _run_apred(adj, hn):
    return pl.pallas_call(
        _apred_body,
        grid=(N // BJ,),
        in_specs=[
            pl.BlockSpec((BJ, N), lambda j: (j, 0)),
            pl.BlockSpec((BJ, 128), lambda j: (j, 0)),
            pl.BlockSpec((N, 128), lambda j: (0, 0)),
        ],
        out_specs=pl.BlockSpec((BJ, N), lambda j: (j, 0)),
        out_shape=jax.ShapeDtypeStruct((N, N), jnp.float32),
    )(adj, hn, hn)


def kernel(x, adj, params):
    p = params
    (x_bar, t1, t2, t3, z, d1, d2, d3, sup1) = _run_ae(x, p)

    fws = [p["f%d_w" % i].reshape(1, -1) for i in range(1, 9)]
    gws = [p["g%d_w" % i] for i in range(1, 9)]
    fuse_t = [t1, t2, t3, z, d1, d2, d3, x_bar]

    h, hn, sup, adjrow_c, adjrow_r = _run_l1(adj, sup1, t1, fws[0], gws[1])

    h1_saved = None
    for li in range(2, 9):
        wn = gws[li] if li < 8 else None
        res = _run_layer(adj, hn, sup, adjrow_c, fuse_t[li - 1],
                         fws[li - 1], wn)
        if li < 8:
            h, hn, sup = res
        else:
            h, hn = res
        if li == 4:
            h1_saved = h

    a_pred = _run_apred(adj, hn)
    return (x_bar, z, a_pred, h, h1_saved)


# speedup vs baseline: 28.1286x; 1.0386x over previous
import jax
import jax.numpy as jnp
from jax import lax
from jax.experimental import pallas as pl
from jax.experimental.pallas import tpu as pltpu

N = 4096
K = 10
MU = 0.5
EPS_NORM = 1e-12
EPS_DEG = 1e-12
NEG = -3.0e38

BJ = 512
BAE = 512

AE_ORDER = ["enc1", "enc2", "enc3", "zl", "dec1", "dec2", "dec3", "xbar"]
AE_RELU = [True, True, True, False, True, True, True, False]


def _dotT(a, b):
    return lax.dot_general(a, b, (((1,), (1,)), ((), ())),
                           preferred_element_type=jnp.float32)


def _dotTL(a, b):
    return lax.dot_general(a, b, (((0,), (0,)), ((), ())),
                           preferred_element_type=jnp.float32)


def _topk_thr(s):
    cur = s
    for _ in range(K - 1):
        m = jnp.max(cur, axis=1, keepdims=True)
        cur = jnp.where(cur >= m, NEG, cur)
    return jnp.max(cur, axis=1, keepdims=True)


def _ae_body(x_ref, *refs):
    w_refs = refs[0:8]
    b_refs = refs[8:16]
    g1_ref = refs[16]
    outs = refs[17:]
    h = x_ref[...]
    hs = []
    for wr, br, rl in zip(w_refs, b_refs, AE_RELU):
        h = jnp.dot(h, wr[...], preferred_element_type=jnp.float32) + br[...]
        if rl:
            h = jax.nn.relu(h)
        hs.append(h)
    outs[0][...] = hs[7]
    outs[1][...] = hs[0]
    outs[2][...] = hs[1]
    outs[3][...] = hs[2]
    outs[4][...] = hs[3]
    outs[5][...] = hs[4]
    outs[6][...] = hs[5]
    outs[7][...] = hs[6]
    outs[8][...] = jnp.dot(x_ref[...], g1_ref[...],
                           preferred_element_type=jnp.float32)


def _run_ae(x, p):
    dims = [128, 64, 32, 16, 32, 64, 128]
    in_specs = [pl.BlockSpec((BAE, 128), lambda i: (i, 0))]
    args = [x]
    for nm in AE_ORDER:
        w = p[nm + "_w"]
        in_specs.append(pl.BlockSpec(w.shape, lambda i: (0, 0)))
        args.append(w)
    for nm in AE_ORDER:
        b = p[nm + "_b"].reshape(1, -1)
        in_specs.append(pl.BlockSpec(b.shape, lambda i: (0, 0)))
        args.append(b)
    in_specs.append(pl.BlockSpec((128, 128), lambda i: (0, 0)))
    args.append(p["g1_w"])
    out_dims = [128] + dims + [128]
    out_shapes = [jax.ShapeDtypeStruct((N, d), jnp.float32) for d in out_dims]
    out_specs = [pl.BlockSpec((BAE, d), lambda i: (i, 0)) for d in out_dims]
    return pl.pallas_call(
        _ae_body,
        grid=(N // BAE,),
        in_specs=in_specs,
        out_specs=out_specs,
        out_shape=out_shapes,
    )(*args)


def _l1_body(adj_ref, sup1_ref, t1_ref, f1_ref, g2_ref,
             h_ref, hn_ref, sup2_ref, arc_ref, arr_ref):
    i = pl.program_id(0)
    acc = jnp.dot(adj_ref[...], sup1_ref[...],
                  preferred_element_type=jnp.float32)
    g = jax.nn.sigmoid(f1_ref[...])
    h = g * jax.nn.relu(acc) + (1.0 - g) * t1_ref[...]
    nrm = jnp.sqrt(jnp.sum(h * h, axis=1, keepdims=True)) + EPS_NORM
    h_ref[...] = h
    hn_ref[...] = h / nrm
    sup2_ref[...] = jnp.dot(h, g2_ref[...], preferred_element_type=jnp.float32)
    rs = jnp.sum(adj_ref[...], axis=1, keepdims=True)
    arc_ref[...] = rs
    arr_ref[0, pl.ds(i * BJ, BJ)] = rs[:, 0]


def _run_l1(adj, sup1, t1, f1w, g2w):
    f2o = g2w.shape[1]
    return pl.pallas_call(
        _l1_body,
        grid=(N // BJ,),
        in_specs=[
            pl.BlockSpec((BJ, N), lambda i: (i, 0)),
            pl.BlockSpec((N, 128), lambda i: (0, 0)),
            pl.BlockSpec((BJ, 128), lambda i: (i, 0)),
            pl.BlockSpec((1, 128), lambda i: (0, 0)),
            pl.BlockSpec((128, f2o), lambda i: (0, 0)),
        ],
        out_specs=[
            pl.BlockSpec((BJ, 128), lambda i: (i, 0)),
            pl.BlockSpec((BJ, 128), lambda i: (i, 0)),
            pl.BlockSpec((BJ, f2o), lambda i: (i, 0)),
            pl.BlockSpec((BJ, 1), lambda i: (i, 0)),
            pl.BlockSpec((1, N), lambda i: (0, 0)),
        ],
        out_shape=[
            jax.ShapeDtypeStruct((N, 128), jnp.float32),
            jax.ShapeDtypeStruct((N, 128), jnp.float32),
            jax.ShapeDtypeStruct((N, f2o), jnp.float32),
            jax.ShapeDtypeStruct((N, 1), jnp.float32),
            jax.ShapeDtypeStruct((1, N), jnp.float32),
        ],
    )(adj, sup1, t1, f1w, g2w)


_G = N // BJ


def _layer_body(adj_ref, hnj_ref, hn_ref, supj_ref, arc_ref, t_ref,
                fw_ref, wn_ref, h_ref, hno_ref, supo_ref,
                thr_s, degc_s, degr_s, hacc_s):
    t = pl.program_id(0)
    i = lax.rem(t, _G)
    s = _dotT(hnj_ref[...], hn_ref[...])

    @pl.when(t < _G)
    def _p1():
        thr = _topk_thr(s)
        st = jnp.where(jnp.logical_and(s >= thr, s > 0.0), s, 0.0)
        srow = jnp.sum(st, axis=1, keepdims=True)
        deg = 1.0 / jnp.sqrt((1.0 - MU) * arc_ref[...] + MU * srow + EPS_DEG)
        thr_s[pl.ds(i * BJ, BJ), :] = thr
        degc_s[pl.ds(i * BJ, BJ), :] = deg
        degr_s[0, pl.ds(i * BJ, BJ)] = deg[:, 0]

    @pl.when(t >= _G)
    def _p2():
        thr = thr_s[pl.ds(i * BJ, BJ), :]
        st = jnp.where(jnp.logical_and(s >= thr, s > 0.0), s, 0.0)
        w = (1.0 - MU) * adj_ref[...] + MU * st
        a = (w * degr_s[...]) * degc_s[pl.ds(i * BJ, BJ), :]
        contrib = _dotTL(a, supj_ref[...])

        @pl.when(t == _G)
        def _():
            hacc_s[...] = jnp.zeros_like(hacc_s)

        hacc_s[...] += contrib

    @pl.when(t == 2 * _G - 1)
    def _epi():
        g = jax.nn.sigmoid(fw_ref[...])
        h = g * jax.nn.relu(hacc_s[...]) + (1.0 - g) * t_ref[...]
        nrm = jnp.sqrt(jnp.sum(h * h, axis=1, keepdims=True)) + EPS_NORM
        h_ref[...] = h
        hno_ref[...] = h / nrm
        if supo_ref is not None:
            supo_ref[...] = jnp.dot(h, wn_ref[...],
                                    preferred_element_type=jnp.float32)


def _layer_body_last(adj_ref, hnj_ref, hn_ref, supj_ref, arc_ref, t_ref,
                     fw_ref, h_ref, hno_ref, thr_s, degc_s, degr_s, hacc_s):
    _layer_body(adj_ref, hnj_ref, hn_ref, supj_ref, arc_ref, t_ref,
                fw_ref, None, h_ref, hno_ref, None,
                thr_s, degc_s, degr_s, hacc_s)


def _run_layer(adj, hn, sup, arc, t_fuse, fw, wn):
    f = hn.shape[1]
    fo = sup.shape[1]
    in_specs = [
        pl.BlockSpec((BJ, N), lambda t: (jnp.maximum(t - _G, 0), 0)),
        pl.BlockSpec((BJ, f), lambda t: (lax.rem(t, _G), 0)),
        pl.BlockSpec((N, f), lambda t: (0, 0)),
        pl.BlockSpec((BJ, fo), lambda t: (lax.rem(t, _G), 0)),
        pl.BlockSpec((BJ, 1), lambda t: (lax.rem(t, _G), 0)),
        pl.BlockSpec((N, fo), lambda t: (0, 0)),
        pl.BlockSpec((1, fo), lambda t: (0, 0)),
    ]
    args = [adj, hn, hn, sup, arc, t_fuse, fw]
    out_shapes = [jax.ShapeDtypeStruct((N, fo), jnp.float32),
                  jax.ShapeDtypeStruct((N, fo), jnp.float32)]
    out_specs = [pl.BlockSpec((N, fo), lambda t: (0, 0)),
                 pl.BlockSpec((N, fo), lambda t: (0, 0))]
    if wn is not None:
        fo2 = wn.shape[1]
        in_specs.append(pl.BlockSpec((f if False else wn.shape[0], fo2),
                                     lambda t: (0, 0)))
        args.append(wn)
        out_shapes.append(jax.ShapeDtypeStruct((N, fo2), jnp.float32))
        out_specs.append(pl.BlockSpec((N, fo2), lambda t: (0, 0)))
        body = _layer_body
    else:
        body = _layer_body_last
    return pl.pallas_call(
        body,
        grid=(2 * _G,),
        in_specs=in_specs,
        out_specs=out_specs,
        out_shape=out_shapes,
        scratch_shapes=[
            pltpu.VMEM((N, 1), jnp.float32),
            pltpu.VMEM((N, 1), jnp.float32),
            pltpu.VMEM((1, N), jnp.float32),
            pltpu.VMEM((N, fo), jnp.float32),
        ],
    )(*args)


def _apred_body(adj_ref, hnj_ref, hn_ref, a_ref):
    s = _dotT(hnj_ref[...], hn_ref[...])
    thr = _topk_thr(s)
    st = jnp.where(jnp.logical_and(s >= thr, s > 0.0), s, 0.0)
    a_ref[...] = (1.0 - MU) * adj_ref[...] + MU * st


def _run_apred(adj, hn):
    return pl.pallas_call(
        _apred_body,
        grid=(N // BJ,),
        in_specs=[
            pl.BlockSpec((BJ, N), lambda j: (j, 0)),
            pl.BlockSpec((BJ, 128), lambda j: (j, 0)),
            pl.BlockSpec((N, 128), lambda j: (0, 0)),
        ],
        out_specs=pl.BlockSpec((BJ, N), lambda j: (j, 0)),
        out_shape=jax.ShapeDtypeStruct((N, N), jnp.float32),
    )(adj, hn, hn)


def kernel(x, adj, params):
    p = params
    (x_bar, t1, t2, t3, z, d1, d2, d3, sup1) = _run_ae(x, p)

    fws = [p["f%d_w" % i].reshape(1, -1) for i in range(1, 9)]
    gws = [p["g%d_w" % i] for i in range(1, 9)]
    fuse_t = [t1, t2, t3, z, d1, d2, d3, x_bar]

    h, hn, sup, adjrow_c, adjrow_r = _run_l1(adj, sup1, t1, fws[0], gws[1])

    h1_saved = None
    for li in range(2, 9):
        wn = gws[li] if li < 8 else None
        res = _run_layer(adj, hn, sup, adjrow_c, fuse_t[li - 1],
                         fws[li - 1], wn)
        if li < 8:
            h, hn, sup = res
        else:
            h, hn = res
        if li == 4:
            h1_saved = h

    a_pred = _run_apred(adj, hn)
    return (x_bar, z, a_pred, h, h1_saved)
